# baseline (speedup 1.0000x reference)
import functools

import jax
import jax.numpy as jnp
from jax import lax
from jax.experimental import pallas as pl
from jax.experimental.pallas import tpu as pltpu
from jax.experimental.pallas import tpu_sc as plsc

_GAMMA = 1.0
_NEG_INF = float("-inf")

_CH = 4096
_NCH = 8
_NW = 32
_LANES = 16

_BC_TC = 4096


def _tc_body(y_ref, x_ref, m_ref, c_ref, *, bc, ctc, nsteps, nrows):
    c = pl.program_id(0)

    @pl.when(c == 0)
    def _init():
        m_ref[...] = jnp.full((nrows, 1), _NEG_INF, dtype=jnp.float32)
        c_ref[...] = jnp.zeros((nrows, 1), dtype=jnp.float32)

    xb = x_ref[...]
    col_ids = c * bc + jax.lax.broadcasted_iota(jnp.int32, (nrows, bc), 1)
    eq = col_ids == y_ref[...]
    oob = col_ids >= ctc
    masked = jnp.where(eq | oob, _NEG_INF, xb)
    m_ref[...] = jnp.maximum(m_ref[...], jnp.max(masked, axis=1, keepdims=True))
    c_ref[...] = c_ref[...] + jnp.sum(
        jnp.where(eq & jnp.logical_not(oob), xb, 0.0), axis=1, keepdims=True
    )


def _sc_body(
    x_hbm, y_hbm, m_out, c_out, y_v, buf, tbuf, m_v, c_v, *, ctc, rpw, tail
):
    cid = lax.axis_index("c")
    sid = lax.axis_index("s")
    wid = sid * 2 + cid
    base = wid * rpw
    tail0 = ctc + _NCH * _CH

    pltpu.sync_copy(y_hbm.at[pl.ds(base, rpw)], y_v.at[pl.ds(0, rpw)])

    def group_body(g, carry):
        row8 = pl.multiple_of(base + g * 8, 8)
        y_vec = y_v[pl.ds(g * 8, _LANES)]
        y_bs = [
            jnp.zeros((_LANES,), dtype=jnp.int32) + y_vec[r] for r in range(8)
        ]

        def fold(bufref, c0, k, mc):
            accs, caccs = mc
            cols = c0 + k * _LANES + lax.iota(jnp.int32, _LANES)
            na, nc = [], []
            for r in range(8):
                v = bufref[r, pl.ds(k * _LANES, _LANES)]
                eq = cols == y_bs[r]
                na.append(jnp.maximum(accs[r], jnp.where(eq, _NEG_INF, v)))
                nc.append(jnp.maximum(caccs[r], jnp.where(eq, v, _NEG_INF)))
            return (tuple(na), tuple(nc))

        def chunk_body(ch, mc):
            c0 = ctc + ch * _CH
            pltpu.sync_copy(x_hbm.at[pl.ds(row8, 8), pl.ds(c0, _CH)], buf)
            return lax.fori_loop(
                0, _CH // _LANES, functools.partial(fold, buf, c0), mc
            )

        neg = jnp.full((_LANES,), _NEG_INF, dtype=jnp.float32)
        mc = (tuple([neg] * 8), tuple([neg] * 8))
        mc = lax.fori_loop(0, _NCH, chunk_body, mc)

        if tail:
            pltpu.sync_copy(
                x_hbm.at[pl.ds(row8, 8), pl.ds(tail0, tail)], tbuf
            )
            for k in range(tail // _LANES):
                mc = fold(tbuf, tail0, k, mc)

        accs, caccs = mc
        for r in range(8):
            m_v[pl.ds((g * 8 + r) * _LANES, _LANES)] = accs[r]
            c_v[pl.ds((g * 8 + r) * _LANES, _LANES)] = caccs[r]
        return carry

    lax.fori_loop(0, rpw // 8, group_body, 0)

    pltpu.sync_copy(m_v, m_out.at[pl.ds(base * _LANES, rpw * _LANES)])
    pltpu.sync_copy(c_v, c_out.at[pl.ds(base * _LANES, rpw * _LANES)])


def _combine_body(mt_ref, ct_ref, ms_ref, cs_ref, o_ref, *, nrows):
    ms = jnp.max(ms_ref[...], axis=1, keepdims=True)
    cv = jnp.max(cs_ref[...], axis=1, keepdims=True)
    cs = jnp.where(cv == _NEG_INF, 0.0, cv)
    m = jnp.maximum(mt_ref[...], ms)
    corr = ct_ref[...] + cs
    loss = _GAMMA + m - corr
    o_ref[0, 0] = jnp.sum(loss) * (1.0 / nrows)


def kernel(x, y):
    nrows, ncols = x.shape
    tail = ncols % 128
    w_sc = _CH * _NCH + tail
    ctc = ncols - w_sc
    rpw = nrows // _NW
    y32 = y.astype(jnp.int32)

    mesh = plsc.VectorSubcoreMesh(core_axis_name="c", subcore_axis_name="s")
    sc_fn = pl.kernel(
        functools.partial(_sc_body, ctc=ctc, rpw=rpw, tail=tail),
        mesh=mesh,
        out_type=[
            jax.ShapeDtypeStruct((nrows * _LANES,), jnp.float32),
            jax.ShapeDtypeStruct((nrows * _LANES,), jnp.float32),
        ],
        scratch_types=[
            pltpu.VMEM((rpw + _LANES,), jnp.int32),
            pltpu.VMEM((8, _CH), jnp.float32),
            pltpu.VMEM((8, max(tail, _LANES)), jnp.float32),
            pltpu.VMEM((rpw * _LANES,), jnp.float32),
            pltpu.VMEM((rpw * _LANES,), jnp.float32),
        ],
    )
    m_sc, c_sc = sc_fn(x, y32)

    nsteps = pl.cdiv(ctc, _BC_TC)
    tc_body = functools.partial(
        _tc_body, bc=_BC_TC, ctc=ctc, nsteps=nsteps, nrows=nrows
    )
    m_tc, c_tc = pl.pallas_call(
        tc_body,
        grid=(nsteps,),
        in_specs=[
            pl.BlockSpec((nrows, 1), lambda c: (0, 0)),
            pl.BlockSpec((nrows, _BC_TC), lambda c: (0, c)),
        ],
        out_specs=[
            pl.BlockSpec((nrows, 1), lambda c: (0, 0)),
            pl.BlockSpec((nrows, 1), lambda c: (0, 0)),
        ],
        out_shape=[
            jax.ShapeDtypeStruct((nrows, 1), jnp.float32),
            jax.ShapeDtypeStruct((nrows, 1), jnp.float32),
        ],
        compiler_params=pltpu.CompilerParams(
            dimension_semantics=("arbitrary",),
        ),
    )(y32.reshape(nrows, 1), x)

    combine = functools.partial(_combine_body, nrows=nrows)
    out = pl.pallas_call(
        combine,
        out_specs=pl.BlockSpec(memory_space=pltpu.SMEM),
        out_shape=jax.ShapeDtypeStruct((1, 1), jnp.float32),
    )(
        m_tc,
        c_tc,
        m_sc.reshape(nrows, _LANES),
        c_sc.reshape(nrows, _LANES),
    )
    return out[0, 0]


# speedup vs baseline: 2.4550x; 2.4550x over previous
import functools

import jax
import jax.numpy as jnp
from jax.experimental import pallas as pl
from jax.experimental.pallas import tpu as pltpu

_GAMMA = 1.0
_NEG_INF = float("-inf")
_BC = 2048


def _tc_body(y_ref, xt_ref, o_ref, m_ref, c_ref, *, bc, ncls, nsteps, nb):
    c = pl.program_id(0)

    @pl.when(c == 0)
    def _init():
        m_ref[...] = jnp.full((1, nb), _NEG_INF, dtype=jnp.float32)
        c_ref[...] = jnp.zeros((1, nb), dtype=jnp.float32)

    xb = xt_ref[...]
    cls_ids = c * bc + jax.lax.broadcasted_iota(jnp.int32, (bc, nb), 0)
    eq = cls_ids == y_ref[...]
    oob = cls_ids >= ncls
    masked = jnp.where(eq | oob, _NEG_INF, xb)
    m_ref[...] = jnp.maximum(m_ref[...], jnp.max(masked, axis=0, keepdims=True))
    c_ref[...] = c_ref[...] + jnp.sum(
        jnp.where(eq & jnp.logical_not(oob), xb, 0.0), axis=0, keepdims=True
    )

    @pl.when(c == nsteps - 1)
    def _fin():
        loss = _GAMMA + m_ref[...] - c_ref[...]
        o_ref[0, 0] = jnp.sum(loss) * (1.0 / nb)


def kernel(x, y):
    nb, ncls = x.shape
    xt = x.T
    nsteps = pl.cdiv(ncls, _BC)
    y2 = y.astype(jnp.int32).reshape(1, nb)

    body = functools.partial(
        _tc_body, bc=_BC, ncls=ncls, nsteps=nsteps, nb=nb
    )
    out = pl.pallas_call(
        body,
        grid=(nsteps,),
        in_specs=[
            pl.BlockSpec((1, nb), lambda c: (0, 0)),
            pl.BlockSpec((_BC, nb), lambda c: (c, 0)),
        ],
        out_specs=pl.BlockSpec(memory_space=pltpu.SMEM),
        out_shape=jax.ShapeDtypeStruct((1, 1), jnp.float32),
        scratch_shapes=[
            pltpu.VMEM((1, nb), jnp.float32),
            pltpu.VMEM((1, nb), jnp.float32),
        ],
        compiler_params=pltpu.CompilerParams(
            dimension_semantics=("arbitrary",),
        ),
    )(y2, xt)
    return out[0, 0]


# speedup vs baseline: 3.4033x; 1.3863x over previous
import functools

import jax
import jax.numpy as jnp
from jax.experimental import pallas as pl
from jax.experimental.pallas import tpu as pltpu

_GAMMA = 1.0
_NEG_INF = float("-inf")
_BC = 2048


def _tc_body(y_ref, xt_ref, o_ref, m_ref, c_ref, *, bc, ncls, nsteps, nb):
    c = pl.program_id(0)

    @pl.when(c == 0)
    def _init():
        m_ref[...] = jnp.full((1, nb), _NEG_INF, dtype=jnp.float32)
        c_ref[...] = jnp.zeros((1, nb), dtype=jnp.float32)

    xb = xt_ref[...]
    li = jax.lax.broadcasted_iota(jnp.int32, (bc, nb), 0)
    y_loc = y_ref[...] - c * bc
    eq = li == y_loc

    @pl.when(c < nsteps - 1)
    def _main():
        masked = jnp.where(eq, _NEG_INF, xb)
        m_ref[...] = jnp.maximum(
            m_ref[...], jnp.max(masked, axis=0, keepdims=True)
        )
        c_ref[...] = c_ref[...] + jnp.sum(
            jnp.where(eq, xb, 0.0), axis=0, keepdims=True
        )

    @pl.when(c == nsteps - 1)
    def _tail():
        oob = li >= (ncls - c * bc)
        masked = jnp.where(eq | oob, _NEG_INF, xb)
        m_ref[...] = jnp.maximum(
            m_ref[...], jnp.max(masked, axis=0, keepdims=True)
        )
        c_ref[...] = c_ref[...] + jnp.sum(
            jnp.where(eq & jnp.logical_not(oob), xb, 0.0),
            axis=0,
            keepdims=True,
        )

    @pl.when(c == nsteps - 1)
    def _fin():
        loss = _GAMMA + m_ref[...] - c_ref[...]
        o_ref[0, 0] = jnp.sum(loss) * (1.0 / nb)


def kernel(x, y):
    nb, ncls = x.shape
    xt = x.T
    nsteps = pl.cdiv(ncls, _BC)
    y2 = y.astype(jnp.int32).reshape(1, nb)

    body = functools.partial(
        _tc_body, bc=_BC, ncls=ncls, nsteps=nsteps, nb=nb
    )
    out = pl.pallas_call(
        body,
        grid=(nsteps,),
        in_specs=[
            pl.BlockSpec((1, nb), lambda c: (0, 0)),
            pl.BlockSpec((_BC, nb), lambda c: (c, 0)),
        ],
        out_specs=pl.BlockSpec(memory_space=pltpu.SMEM),
        out_shape=jax.ShapeDtypeStruct((1, 1), jnp.float32),
        scratch_shapes=[
            pltpu.VMEM((1, nb), jnp.float32),
            pltpu.VMEM((1, nb), jnp.float32),
        ],
        compiler_params=pltpu.CompilerParams(
            dimension_semantics=("arbitrary",),
        ),
    )(y2, xt)
    return out[0, 0]
